# baseline (speedup 1.0000x reference)
import functools

import jax
import jax.numpy as jnp
from jax import lax
from jax.experimental import pallas as pl
from jax.experimental.pallas import tpu as pltpu
from jax.experimental.pallas import tpu_sc as plsc

N_NODES = 10000
D = 128
N_EDGES = 320000

NC = 2
NS = 16
NW = NC * NS
CHUNK = 64
BC = 8
NBLK = 20
EDGES_PER_W = NBLK * BC * CHUNK
N_EDGES_PAD = EDGES_PER_W * NW
ROWS_PER_TILE = 640
N_PAD = ROWS_PER_TILE * NS


def _mm_body(x_ref, w_ref, b_ref, h_ref):
    h_ref[...] = (
        jnp.dot(x_ref[...], w_ref[...], preferred_element_type=jnp.float32)
        + b_ref[...]
    )


def _matmul(x, W, b2d):
    return pl.pallas_call(
        _mm_body,
        out_shape=jax.ShapeDtypeStruct((N_NODES, D), jnp.float32),
    )(x, W, b2d)


def _sc_agg(h, src_r, dst_r, zeros):
    mesh = plsc.VectorSubcoreMesh(core_axis_name="c", subcore_axis_name="s")

    @functools.partial(
        pl.kernel,
        mesh=mesh,
        out_type=jax.ShapeDtypeStruct((NC, N_PAD, D), jnp.float32),
        scratch_types=[
            pltpu.VMEM((BC, CHUNK), jnp.int32),
            pltpu.VMEM((BC, CHUNK), jnp.int32),
            pltpu.VMEM((CHUNK, D), jnp.float32),
            pltpu.VMEM((CHUNK, D), jnp.float32),
            pltpu.VMEM_SHARED((N_PAD, D), jnp.float32),
            pltpu.SemaphoreType.DMA,
            pltpu.SemaphoreType.DMA,
            pltpu.SemaphoreType.DMA,
            pltpu.SemaphoreType.DMA,
            pltpu.SemaphoreType.DMA,
        ],
    )
    def k(h_hbm, src_hbm, dst_hbm, z_hbm, out_hbm,
          src_v, dst_v, rows0, rows1, acc, gi, g0, g1, s0, s1):
        c = lax.axis_index("c")
        s = lax.axis_index("s")
        wid = c * NS + s

        pltpu.sync_copy(z_hbm, acc.at[pl.ds(s * ROWS_PER_TILE, ROWS_PER_TILE)])
        plsc.subcore_barrier()

        def gather(i, buf, sem):
            return pltpu.make_async_copy(h_hbm.at[src_v.at[i]], buf, sem)

        def scat(i, buf, sem):
            return pltpu.make_async_copy(buf, acc.at[dst_v.at[i]], sem)

        def block(b, carry):
            pltpu.make_async_copy(src_hbm.at[wid, b], src_v, gi).start()
            pltpu.make_async_copy(dst_hbm.at[wid, b], dst_v, gi).start()
            pltpu.make_async_copy(src_hbm.at[wid, b], src_v, gi).wait()
            pltpu.make_async_copy(dst_hbm.at[wid, b], dst_v, gi).wait()
            for i in range(BC):
                gather(i, rows0, g0).start()
                gather(i, rows0, g0).wait()
                scat(i, rows0, s0).start(add=True)
                scat(i, rows0, s0).wait()
            return carry

        lax.fori_loop(0, NBLK, block, 0)

        plsc.subcore_barrier()
        pltpu.sync_copy(
            acc.at[pl.ds(s * ROWS_PER_TILE, ROWS_PER_TILE)],
            out_hbm.at[c, pl.ds(s * ROWS_PER_TILE, ROWS_PER_TILE)],
        )

    return k(h, src_r, dst_r, zeros)


def _finish_body(p_ref, g_ref, be_ref, al_ref, w2_ref, b2_ref, o_ref):
    row = lax.broadcasted_iota(jnp.int32, (N_PAD, 1), 0)
    valid = row < N_NODES
    agg = jnp.where(valid, p_ref[0] + p_ref[1], 0.0)
    n = float(N_NODES)
    mean = jnp.sum(agg, axis=0, keepdims=True) / n
    e2 = jnp.sum(agg * agg, axis=0, keepdims=True) / n
    var = e2 - mean * mean
    scale = g_ref[...] * lax.rsqrt(var + 1e-3)
    hn = (agg - mean) * scale + be_ref[...]
    act = jnp.where(hn > 0, hn, al_ref[...] * hn)
    act = jnp.where(valid, act, 0.0)
    pooled = jnp.sum(act, axis=0, keepdims=True)
    o_ref[...] = jnp.sum(pooled * w2_ref[...], keepdims=True) + b2_ref[...]


def _finish(partials, gamma, beta, alpha, W2t, b2):
    return pl.pallas_call(
        _finish_body,
        out_shape=jax.ShapeDtypeStruct((1, 1), jnp.float32),
    )(partials, gamma, beta, alpha, W2t, b2)


def kernel(x, edge_index, W, b, gamma, beta, alpha, W2, b2):
    h = _matmul(x, W, b.reshape(1, D))
    n_fill = N_EDGES_PAD - N_EDGES
    src_pad = jnp.concatenate(
        [edge_index[0], jnp.zeros((n_fill,), jnp.int32)])
    dst_pad = jnp.concatenate(
        [edge_index[1],
         N_NODES + (jnp.arange(n_fill, dtype=jnp.int32) % (N_PAD - N_NODES))])
    src_r = src_pad.reshape(NW, NBLK, BC, CHUNK)
    dst_r = dst_pad.reshape(NW, NBLK, BC, CHUNK)
    zeros = jnp.zeros((ROWS_PER_TILE, D), jnp.float32)
    partials = _sc_agg(h, src_r, dst_r, zeros)
    out = _finish(
        partials,
        gamma.reshape(1, D),
        beta.reshape(1, D),
        alpha.reshape(1, D),
        W2.reshape(1, D),
        b2.reshape(1, 1),
    )
    return out.reshape(1)


# speedup vs baseline: 1.5062x; 1.5062x over previous
import functools

import jax
import jax.numpy as jnp
from jax import lax
from jax.experimental import pallas as pl
from jax.experimental.pallas import tpu as pltpu
from jax.experimental.pallas import tpu_sc as plsc

N_NODES = 10000
D = 128
N_EDGES = 320000

NC = 2
NS = 16
NW = NC * NS
CHUNK = 80
HALF = 64
NH = 2
EDGES_PER_W = NH * HALF * CHUNK
N_EDGES_PAD = EDGES_PER_W * NW
ROWS_PER_TILE = 640
N_PAD = ROWS_PER_TILE * NS


def _mm_body(x_ref, w_ref, b_ref, h_ref):
    h_ref[...] = (
        jnp.dot(x_ref[...], w_ref[...], preferred_element_type=jnp.float32)
        + b_ref[...]
    )


def _matmul(x, W, b2d):
    return pl.pallas_call(
        _mm_body,
        out_shape=jax.ShapeDtypeStruct((N_NODES, D), jnp.float32),
    )(x, W, b2d)


def _sc_agg(h, src_r, dst_r, zeros):
    mesh = plsc.VectorSubcoreMesh(core_axis_name="c", subcore_axis_name="s")

    @functools.partial(
        pl.kernel,
        mesh=mesh,
        out_type=jax.ShapeDtypeStruct((NC, N_PAD, D), jnp.float32),
        scratch_types=[
            pltpu.VMEM((HALF, CHUNK), jnp.int32),
            pltpu.VMEM((HALF, CHUNK), jnp.int32),
            pltpu.VMEM((CHUNK, D), jnp.float32),
            pltpu.VMEM((CHUNK, D), jnp.float32),
            pltpu.VMEM_SHARED((N_PAD, D), jnp.float32),
            pltpu.SemaphoreType.DMA,
            pltpu.SemaphoreType.DMA,
            pltpu.SemaphoreType.DMA,
            pltpu.SemaphoreType.DMA,
            pltpu.SemaphoreType.DMA,
        ],
    )
    def k(h_hbm, src_hbm, dst_hbm, z_hbm, out_hbm,
          src_v, dst_v, rows0, rows1, acc, gi, g0, g1, s0, s1):
        c = lax.axis_index("c")
        s = lax.axis_index("s")
        wid = c * NS + s

        pltpu.sync_copy(z_hbm, acc.at[pl.ds(s * ROWS_PER_TILE, ROWS_PER_TILE)])
        plsc.subcore_barrier()

        def gather(i, buf, sem):
            return pltpu.make_async_copy(h_hbm.at[src_v.at[i]], buf, sem)

        def scat(i, buf, sem):
            return pltpu.make_async_copy(buf, acc.at[dst_v.at[i]], sem)

        for hh in range(NH):
            pltpu.make_async_copy(src_hbm.at[wid, hh], src_v, gi).start()
            pltpu.make_async_copy(dst_hbm.at[wid, hh], dst_v, gi).start()
            pltpu.make_async_copy(src_hbm.at[wid, hh], src_v, gi).wait()
            pltpu.make_async_copy(dst_hbm.at[wid, hh], dst_v, gi).wait()

            gather(0, rows0, g0).start()
            gather(1, rows1, g1).start()

            def pair(p, carry):
                i0 = 2 * p
                i1 = i0 + 1
                gather(i0, rows0, g0).wait()
                scat(i0, rows0, s0).start(add=True)
                gather(i1, rows1, g1).wait()
                scat(i1, rows1, s1).start(add=True)
                scat(i0, rows0, s0).wait()
                gather(i0 + 2, rows0, g0).start()
                scat(i1, rows1, s1).wait()
                gather(i1 + 2, rows1, g1).start()
                return carry

            lax.fori_loop(0, HALF // 2 - 1, pair, 0)

            last0 = HALF - 2
            last1 = HALF - 1
            gather(last0, rows0, g0).wait()
            scat(last0, rows0, s0).start(add=True)
            gather(last1, rows1, g1).wait()
            scat(last1, rows1, s1).start(add=True)
            scat(last0, rows0, s0).wait()
            scat(last1, rows1, s1).wait()

        plsc.subcore_barrier()
        pltpu.sync_copy(
            acc.at[pl.ds(s * ROWS_PER_TILE, ROWS_PER_TILE)],
            out_hbm.at[c, pl.ds(s * ROWS_PER_TILE, ROWS_PER_TILE)],
        )

    return k(h, src_r, dst_r, zeros)


def _finish_body(p_ref, g_ref, be_ref, al_ref, w2_ref, b2_ref, o_ref):
    row = lax.broadcasted_iota(jnp.int32, (N_PAD, 1), 0)
    valid = row < N_NODES
    agg = jnp.where(valid, p_ref[0] + p_ref[1], 0.0)
    n = float(N_NODES)
    mean = jnp.sum(agg, axis=0, keepdims=True) / n
    e2 = jnp.sum(agg * agg, axis=0, keepdims=True) / n
    var = e2 - mean * mean
    scale = g_ref[...] * lax.rsqrt(var + 1e-3)
    hn = (agg - mean) * scale + be_ref[...]
    act = jnp.where(hn > 0, hn, al_ref[...] * hn)
    act = jnp.where(valid, act, 0.0)
    pooled = jnp.sum(act, axis=0, keepdims=True)
    o_ref[...] = jnp.sum(pooled * w2_ref[...], keepdims=True) + b2_ref[...]


def _finish(partials, gamma, beta, alpha, W2t, b2):
    return pl.pallas_call(
        _finish_body,
        out_shape=jax.ShapeDtypeStruct((1, 1), jnp.float32),
    )(partials, gamma, beta, alpha, W2t, b2)


def kernel(x, edge_index, W, b, gamma, beta, alpha, W2, b2):
    h = _matmul(x, W, b.reshape(1, D))
    n_fill = N_EDGES_PAD - N_EDGES
    src_pad = jnp.concatenate(
        [edge_index[0], jnp.zeros((n_fill,), jnp.int32)])
    dst_pad = jnp.concatenate(
        [edge_index[1],
         N_NODES + (jnp.arange(n_fill, dtype=jnp.int32) % (N_PAD - N_NODES))])
    src_r = src_pad.reshape(NW, NH, HALF, CHUNK)
    dst_r = dst_pad.reshape(NW, NH, HALF, CHUNK)
    zeros = jnp.zeros((ROWS_PER_TILE, D), jnp.float32)
    partials = _sc_agg(h, src_r, dst_r, zeros)
    out = _finish(
        partials,
        gamma.reshape(1, D),
        beta.reshape(1, D),
        alpha.reshape(1, D),
        W2.reshape(1, D),
        b2.reshape(1, 1),
    )
    return out.reshape(1)


# speedup vs baseline: 1.5064x; 1.0001x over previous
import functools

import jax
import jax.numpy as jnp
from jax import lax
from jax.experimental import pallas as pl
from jax.experimental.pallas import tpu as pltpu
from jax.experimental.pallas import tpu_sc as plsc

N_NODES = 10000
D = 128
N_EDGES = 320000

NC = 2
NS = 16
NW = NC * NS
CHUNK = 80
HALF = 64
NH = 2
EDGES_PER_W = NH * HALF * CHUNK
N_EDGES_PAD = EDGES_PER_W * NW
ROWS_PER_TILE = 640
N_PAD = ROWS_PER_TILE * NS


def _mm_body(x_ref, w_ref, b_ref, h_ref):
    h_ref[...] = (
        jnp.dot(x_ref[...], w_ref[...], preferred_element_type=jnp.float32)
        + b_ref[...]
    )


def _matmul(x, W, b2d):
    return pl.pallas_call(
        _mm_body,
        out_shape=jax.ShapeDtypeStruct((N_NODES, D), jnp.float32),
    )(x, W, b2d)


def _sc_agg(h, src_r, dst_r, zeros):
    mesh = plsc.VectorSubcoreMesh(core_axis_name="c", subcore_axis_name="s")

    @functools.partial(
        pl.kernel,
        mesh=mesh,
        out_type=jax.ShapeDtypeStruct((NC, N_PAD, D), jnp.float32),
        scratch_types=[
            pltpu.VMEM((HALF, CHUNK), jnp.int32),
            pltpu.VMEM((HALF, CHUNK), jnp.int32),
            pltpu.VMEM((CHUNK, D), jnp.float32),
            pltpu.VMEM((CHUNK, D), jnp.float32),
            pltpu.VMEM_SHARED((N_PAD, D), jnp.float32),
            pltpu.SemaphoreType.DMA,
            pltpu.SemaphoreType.DMA,
            pltpu.SemaphoreType.DMA,
            pltpu.SemaphoreType.DMA,
            pltpu.SemaphoreType.DMA,
        ],
    )
    def k(h_hbm, src_hbm, dst_hbm, z_hbm, out_hbm,
          src_v, dst_v, rows0, rows1, acc, gi, g0, g1, s0, s1):
        c = lax.axis_index("c")
        s = lax.axis_index("s")
        wid = c * NS + s

        pltpu.sync_copy(z_hbm, acc.at[pl.ds(s * ROWS_PER_TILE, ROWS_PER_TILE)])
        plsc.subcore_barrier()

        def gather(i, buf, sem):
            return pltpu.make_async_copy(h_hbm.at[src_v.at[i]], buf, sem)

        def scat(i, buf, sem):
            return pltpu.make_async_copy(buf, acc.at[dst_v.at[i]], sem)

        for hh in range(NH):
            pltpu.make_async_copy(src_hbm.at[wid, hh], src_v, gi).start()
            pltpu.make_async_copy(dst_hbm.at[wid, hh], dst_v, gi).start()
            pltpu.make_async_copy(src_hbm.at[wid, hh], src_v, gi).wait()
            pltpu.make_async_copy(dst_hbm.at[wid, hh], dst_v, gi).wait()

            gather(0, rows0, g0).start()
            gather(1, rows1, g1).start()

            def pair(p, carry):
                i0 = 2 * p
                i1 = i0 + 1
                gather(i0, rows0, g0).wait()
                scat(i0, rows0, s0).start(add=True)
                gather(i1, rows1, g1).wait()
                scat(i1, rows1, s1).start(add=True)
                scat(i0, rows0, s0).wait()
                gather(i0 + 2, rows0, g0).start()
                scat(i1, rows1, s1).wait()
                gather(i1 + 2, rows1, g1).start()
                return carry

            lax.fori_loop(0, HALF // 2 - 1, pair, 0)

            last0 = HALF - 2
            last1 = HALF - 1
            gather(last0, rows0, g0).wait()
            scat(last0, rows0, s0).start(add=True)
            gather(last1, rows1, g1).wait()
            scat(last1, rows1, s1).start(add=True)
            scat(last0, rows0, s0).wait()
            scat(last1, rows1, s1).wait()

        plsc.subcore_barrier()
        pltpu.sync_copy(
            acc.at[pl.ds(s * ROWS_PER_TILE, ROWS_PER_TILE)],
            out_hbm.at[c, pl.ds(s * ROWS_PER_TILE, ROWS_PER_TILE)],
        )

    return k(h, src_r, dst_r, zeros)


def _finish_body(p_ref, g_ref, be_ref, al_ref, w2_ref, b2_ref, o_ref):
    row = lax.broadcasted_iota(jnp.int32, (N_PAD, 1), 0)
    valid = row < N_NODES
    agg = jnp.where(valid, p_ref[0] + p_ref[1], 0.0)
    n = float(N_NODES)
    mean = jnp.sum(agg, axis=0, keepdims=True) / n
    e2 = jnp.sum(agg * agg, axis=0, keepdims=True) / n
    var = e2 - mean * mean
    scale = g_ref[...] * lax.rsqrt(var + 1e-3)
    hn = (agg - mean) * scale + be_ref[...]
    act = jnp.where(hn > 0, hn, al_ref[...] * hn)
    act = jnp.where(valid, act, 0.0)
    pooled = jnp.sum(act, axis=0, keepdims=True)
    o_ref[...] = jnp.sum(pooled * w2_ref[...], keepdims=True) + b2_ref[...]


def _finish(partials, gamma, beta, alpha, W2t, b2):
    return pl.pallas_call(
        _finish_body,
        out_shape=jax.ShapeDtypeStruct((1, 1), jnp.float32),
    )(partials, gamma, beta, alpha, W2t, b2)


def kernel(x, edge_index, W, b, gamma, beta, alpha, W2, b2):
    h = _matmul(x, W, b.reshape(1, D))
    fill_per_w = EDGES_PER_W - N_EDGES // NW
    src_pad = jnp.concatenate(
        [edge_index[0].reshape(NW, N_EDGES // NW),
         jnp.zeros((NW, fill_per_w), jnp.int32)], axis=1)
    dst_fill = N_NODES + (
        jnp.arange(fill_per_w, dtype=jnp.int32) % (N_PAD - N_NODES))
    dst_pad = jnp.concatenate(
        [edge_index[1].reshape(NW, N_EDGES // NW),
         jnp.broadcast_to(dst_fill, (NW, fill_per_w))], axis=1)
    src_r = src_pad.reshape(NW, NH, HALF, CHUNK)
    dst_r = dst_pad.reshape(NW, NH, HALF, CHUNK)
    zeros = jnp.zeros((ROWS_PER_TILE, D), jnp.float32)
    partials = _sc_agg(h, src_r, dst_r, zeros)
    out = _finish(
        partials,
        gamma.reshape(1, D),
        beta.reshape(1, D),
        alpha.reshape(1, D),
        W2.reshape(1, D),
        b2.reshape(1, 1),
    )
    return out.reshape(1)


# speedup vs baseline: 3.5360x; 2.3474x over previous
import functools

import jax
import jax.numpy as jnp
from jax import lax
from jax.experimental import pallas as pl
from jax.experimental.pallas import tpu as pltpu
from jax.experimental.pallas import tpu_sc as plsc

N_NODES = 10000
D = 128
N_EDGES = 320000

NC = 2
NS = 16
NW = NC * NS
CHUNK = 80
HALF = 64
NH = 2
EDGES_PER_W = NH * HALF * CHUNK
N_EDGES_PAD = EDGES_PER_W * NW
ROWS_PER_TILE = 640
N_PAD = ROWS_PER_TILE * NS


def _mm_body(x_ref, w_ref, b_ref, h_ref):
    h_ref[...] = (
        jnp.dot(x_ref[...], w_ref[...], preferred_element_type=jnp.float32)
        + b_ref[...]
    )


def _matmul(x, W, b2d):
    return pl.pallas_call(
        _mm_body,
        out_shape=jax.ShapeDtypeStruct((N_NODES, D), jnp.float32),
    )(x, W, b2d)


def _sc_agg(h, src_r, dst_r, zeros):
    mesh = plsc.VectorSubcoreMesh(core_axis_name="c", subcore_axis_name="s")

    @functools.partial(
        pl.kernel,
        mesh=mesh,
        out_type=jax.ShapeDtypeStruct((NC, N_PAD, D), jnp.float32),
        scratch_types=[
            pltpu.VMEM((HALF, CHUNK), jnp.int32),
            pltpu.VMEM((HALF, CHUNK), jnp.int32),
            pltpu.VMEM((CHUNK, D), jnp.float32),
            pltpu.VMEM((CHUNK, D), jnp.float32),
            pltpu.VMEM_SHARED((N_PAD, D), jnp.float32),
            pltpu.SemaphoreType.DMA,
            pltpu.SemaphoreType.DMA,
            pltpu.SemaphoreType.DMA,
            pltpu.SemaphoreType.DMA,
            pltpu.SemaphoreType.DMA,
        ],
    )
    def k(h_hbm, src_hbm, dst_hbm, z_hbm, out_hbm,
          src_v, dst_v, rows0, rows1, acc, gi, g0, g1, s0, s1):
        c = lax.axis_index("c")
        s = lax.axis_index("s")
        wid = c * NS + s

        pltpu.sync_copy(z_hbm, acc.at[pl.ds(s * ROWS_PER_TILE, ROWS_PER_TILE)])
        plsc.subcore_barrier()

        def gather(i, buf, sem):
            return pltpu.make_async_copy(h_hbm.at[src_v.at[i]], buf, sem)

        def scat(i, buf, sem):
            return pltpu.make_async_copy(buf, acc.at[dst_v.at[i]], sem)

        for hh in range(NH):
            pltpu.make_async_copy(src_hbm.at[wid, hh], src_v, gi).start()
            pltpu.make_async_copy(dst_hbm.at[wid, hh], dst_v, gi).start()
            pltpu.make_async_copy(src_hbm.at[wid, hh], src_v, gi).wait()
            pltpu.make_async_copy(dst_hbm.at[wid, hh], dst_v, gi).wait()

            gather(0, rows0, g0).start()
            gather(1, rows1, g1).start()

            def pair(p, carry):
                i0 = 2 * p
                i1 = i0 + 1
                gather(i0, rows0, g0).wait()
                scat(i0, rows0, s0).start(add=True)
                gather(i1, rows1, g1).wait()
                scat(i1, rows1, s1).start(add=True)
                scat(i0, rows0, s0).wait()
                gather(i0 + 2, rows0, g0).start()
                scat(i1, rows1, s1).wait()
                gather(i1 + 2, rows1, g1).start()
                return carry

            lax.fori_loop(0, HALF // 2 - 1, pair, 0)

            last0 = HALF - 2
            last1 = HALF - 1
            gather(last0, rows0, g0).wait()
            scat(last0, rows0, s0).start(add=True)
            gather(last1, rows1, g1).wait()
            scat(last1, rows1, s1).start(add=True)
            scat(last0, rows0, s0).wait()
            scat(last1, rows1, s1).wait()

        plsc.subcore_barrier()
        pltpu.sync_copy(
            acc.at[pl.ds(s * ROWS_PER_TILE, ROWS_PER_TILE)],
            out_hbm.at[c, pl.ds(s * ROWS_PER_TILE, ROWS_PER_TILE)],
        )

    return k(h, src_r, dst_r, zeros)


def _finish_body(p_ref, g_ref, be_ref, al_ref, w2_ref, b2_ref, o_ref):
    row = lax.broadcasted_iota(jnp.int32, (N_PAD, 1), 0)
    valid = row < N_NODES
    agg = jnp.where(valid, p_ref[0] + p_ref[1], 0.0)
    n = float(N_NODES)
    mean = jnp.sum(agg, axis=0, keepdims=True) / n
    e2 = jnp.sum(agg * agg, axis=0, keepdims=True) / n
    var = e2 - mean * mean
    scale = g_ref[...] * lax.rsqrt(var + 1e-3)
    hn = (agg - mean) * scale + be_ref[...]
    act = jnp.where(hn > 0, hn, al_ref[...] * hn)
    act = jnp.where(valid, act, 0.0)
    pooled = jnp.sum(act, axis=0, keepdims=True)
    o_ref[...] = jnp.sum(pooled * w2_ref[...], keepdims=True) + b2_ref[...]


def _finish(partials, gamma, beta, alpha, W2t, b2):
    return pl.pallas_call(
        _finish_body,
        out_shape=jax.ShapeDtypeStruct((1, 1), jnp.float32),
    )(partials, gamma, beta, alpha, W2t, b2)


def kernel(x, edge_index, W, b, gamma, beta, alpha, W2, b2):
    h = _matmul(x, W, b.reshape(1, D))
    fill_per_w = EDGES_PER_W - N_EDGES // NW
    src_fill = jnp.arange(fill_per_w, dtype=jnp.int32) % N_NODES
    src_pad = jnp.concatenate(
        [edge_index[0].reshape(NW, N_EDGES // NW),
         jnp.broadcast_to(src_fill, (NW, fill_per_w))], axis=1)
    dst_fill = N_NODES + (
        jnp.arange(fill_per_w, dtype=jnp.int32) % (N_PAD - N_NODES))
    dst_pad = jnp.concatenate(
        [edge_index[1].reshape(NW, N_EDGES // NW),
         jnp.broadcast_to(dst_fill, (NW, fill_per_w))], axis=1)
    src_r = src_pad.reshape(NW, NH, HALF, CHUNK)
    dst_r = dst_pad.reshape(NW, NH, HALF, CHUNK)
    zeros = jnp.zeros((ROWS_PER_TILE, D), jnp.float32)
    partials = _sc_agg(h, src_r, dst_r, zeros)
    out = _finish(
        partials,
        gamma.reshape(1, D),
        beta.reshape(1, D),
        alpha.reshape(1, D),
        W2.reshape(1, D),
        b2.reshape(1, 1),
    )
    return out.reshape(1)
